# baseline (speedup 1.0000x reference)
import functools

import jax
import jax.numpy as jnp
from jax import lax
from jax.experimental import pallas as pl
from jax.experimental.pallas import tpu as pltpu
from jax.experimental.pallas import tpu_sc as plsc

B = 16384
EMB = 64

_SC_INFO = plsc.get_sparse_core_info()
_NC = _SC_INFO.num_cores
_NS = _SC_INFO.num_subcores
_NW = _NC * _NS
_BPW = B // _NW


def _sc_gather_body(ids_p_hbm, ids_c_hbm, gmf_pe_hbm, gmf_ce_hbm,
                    mlp_pe_hbm, mlp_ce_hbm,
                    gmf_p_out, gmf_c_out, mlp_p_out, mlp_c_out,
                    idx_p, idx_c, buf_a, buf_b, sem_a, sem_b):
    wid = lax.axis_index("s") * _NC + lax.axis_index("c")
    base = wid * _BPW
    pltpu.sync_copy(ids_p_hbm.at[pl.ds(base, _BPW)], idx_p)
    pltpu.sync_copy(ids_c_hbm.at[pl.ds(base, _BPW)], idx_c)

    cp_a = pltpu.async_copy(gmf_pe_hbm.at[idx_p], buf_a, sem_a)
    cp_b = pltpu.async_copy(gmf_ce_hbm.at[idx_c], buf_b, sem_b)
    cp_a.wait()
    pltpu.sync_copy(buf_a, gmf_p_out.at[pl.ds(base, _BPW)])
    cp_b.wait()
    pltpu.sync_copy(buf_b, gmf_c_out.at[pl.ds(base, _BPW)])

    cp_a = pltpu.async_copy(mlp_pe_hbm.at[idx_p], buf_a, sem_a)
    cp_b = pltpu.async_copy(mlp_ce_hbm.at[idx_c], buf_b, sem_b)
    cp_a.wait()
    pltpu.sync_copy(buf_a, mlp_p_out.at[pl.ds(base, _BPW)])
    cp_b.wait()
    pltpu.sync_copy(buf_b, mlp_c_out.at[pl.ds(base, _BPW)])


_row_sds = jax.ShapeDtypeStruct((B, EMB), jnp.float32)

_sc_gather = pl.kernel(
    _sc_gather_body,
    out_type=[_row_sds, _row_sds, _row_sds, _row_sds],
    mesh=plsc.VectorSubcoreMesh(core_axis_name="c", subcore_axis_name="s"),
    scratch_types=[
        pltpu.VMEM((_BPW,), jnp.int32),
        pltpu.VMEM((_BPW,), jnp.int32),
        pltpu.VMEM((_BPW, EMB), jnp.float32),
        pltpu.VMEM((_BPW, EMB), jnp.float32),
        pltpu.SemaphoreType.DMA,
        pltpu.SemaphoreType.DMA,
    ],
)


_TB = 1024


def _mlp_kernel(gmf_p, gmf_c, mlp_p, mlp_c, w1t, b1, w2t, b2, w3t, b3,
                wp_g, wp_h, bp, out_ref):
    f32 = jnp.float32
    hi = jax.lax.Precision.HIGHEST
    xp = mlp_p[...]
    xc = mlp_c[...]
    h = jnp.dot(xp, w1t[:EMB, :], preferred_element_type=f32, precision=hi)
    h += jnp.dot(xc, w1t[EMB:, :], preferred_element_type=f32, precision=hi)
    h = jnp.maximum(h + b1[...], 0.0)
    h = jnp.maximum(
        jnp.dot(h, w2t[...], preferred_element_type=f32, precision=hi) + b2[...], 0.0)
    h = jnp.maximum(
        jnp.dot(h, w3t[...], preferred_element_type=f32, precision=hi) + b3[...], 0.0)
    g = gmf_p[...] * gmf_c[...]
    logit = jnp.dot(g, wp_g[...], preferred_element_type=f32, precision=hi)
    logit += jnp.dot(h, wp_h[...], preferred_element_type=f32, precision=hi)
    out_ref[...] = jax.nn.sigmoid(logit + bp[...])


def _full(shape):
    return pl.BlockSpec(shape, lambda i: (0, 0))


def kernel(player_ids, champion_ids, gmf_pe, gmf_ce, mlp_pe, mlp_ce,
           W1, b1, W2, b2, W3, b3, Wp, bp):
    ids_p = player_ids.astype(jnp.int32)
    ids_c = champion_ids.astype(jnp.int32)

    gmf_p = jnp.take(gmf_pe, ids_p, axis=0)
    gmf_c = jnp.take(gmf_ce, ids_c, axis=0)
    mlp_p = jnp.take(mlp_pe, ids_p, axis=0)
    mlp_c = jnp.take(mlp_ce, ids_c, axis=0)

    w1t = W1.T
    w2t = W2.T
    w3t = W3.T
    wp_g = Wp[:, :EMB].T
    wp_h = Wp[:, EMB:].T
    b1r = b1.reshape(1, -1)
    b2r = b2.reshape(1, -1)
    b3r = b3.reshape(1, -1)
    bpr = bp.reshape(1, 1)

    row_spec = pl.BlockSpec((_TB, EMB), lambda i: (i, 0))
    out = pl.pallas_call(
        _mlp_kernel,
        grid=(B // _TB,),
        in_specs=[
            row_spec, row_spec, row_spec, row_spec,
            _full((128, 128)), _full((1, 128)),
            _full((128, 64)), _full((1, 64)),
            _full((64, 32)), _full((1, 32)),
            _full((64, 1)), _full((32, 1)), _full((1, 1)),
        ],
        out_specs=pl.BlockSpec((_TB, 1), lambda i: (i, 0)),
        out_shape=jax.ShapeDtypeStruct((B, 1), jnp.float32),
    )(gmf_p, gmf_c, mlp_p, mlp_c, w1t, b1r, w2t, b2r, w3t, b3r,
      wp_g, wp_h, bpr)
    return out


# speedup vs baseline: 1.2431x; 1.2431x over previous
import jax
import jax.numpy as jnp
from jax import lax
from jax.experimental import pallas as pl
from jax.experimental.pallas import tpu as pltpu
from jax.experimental.pallas import tpu_sc as plsc

B = 16384
EMB = 64
NPLAYERS = 1000000
NCHAMPS = 1000

_SC_INFO = plsc.get_sparse_core_info()
_NC = _SC_INFO.num_cores
_NS = _SC_INFO.num_subcores
_NW = _NC * _NS
_BPW = B // _NW

_CW = 128
_TAIL_LO = (NPLAYERS // _CW) * _CW
_NTAIL = NPLAYERS - _TAIL_LO
_BIG = 0x3FFFFFFF


def _sc_gather_body(sorted_hbm, order_hbm, gmf_peT, mlp_peT,
                    gmf_tail_hbm, mlp_tail_hbm,
                    out_a_hbm, out_b_hbm,
                    idx_v, ord_v,
                    chk_a, chk_b, tail_a, tail_b, stg_a, stg_b,
                    sem_a, sem_b, sem_oa, sem_ob):
    wid = lax.axis_index("s") * _NC + lax.axis_index("c")
    base = wid * _BPW
    pltpu.sync_copy(sorted_hbm.at[pl.ds(base, _BPW)], idx_v)
    pltpu.sync_copy(order_hbm.at[pl.ds(base, _BPW)], ord_v)
    pltpu.sync_copy(gmf_tail_hbm, tail_a)
    pltpu.sync_copy(mlp_tail_hbm, tail_b)

    lanes = lax.iota(jnp.int32, 16)
    chunk0 = gmf_peT.at[:, pl.ds(0, _CW)]

    def group_body(g, cur):
        ids16 = idx_v[pl.ds(g * 16, 16)]
        ord16 = ord_v[pl.ds(g * 16, 16)]
        goff = g * 16 * EMB
        for j in range(16):
            idx = ids16[j]
            pos = ord16[j]
            is_tail = idx >= _TAIL_LO
            c = lax.shift_right_logical(idx, 7)
            need = jnp.logical_and(jnp.logical_not(is_tail), c != cur)

            @pl.when(need)
            def _(c=c):
                cc = pl.multiple_of(c * _CW, _CW)
                pltpu.async_copy(gmf_peT.at[:, pl.ds(cc, _CW)], chk_a, sem_a)
                pltpu.async_copy(mlp_peT.at[:, pl.ds(cc, _CW)], chk_b, sem_b)
                pltpu.make_async_copy(chunk0, chk_a, sem_a).wait()
                pltpu.make_async_copy(chunk0, chk_b, sem_b).wait()

            @pl.when(jnp.logical_not(is_tail))
            def _(idx=idx, c=c, j=j):
                col = jnp.full((16,), idx - c * _CW, dtype=jnp.int32)
                for gg in range(4):
                    rows = lanes + (gg * 16)
                    stg_a[pl.ds(goff + j * EMB + gg * 16, 16)] = \
                        plsc.load_gather(chk_a, [rows, col])
                    stg_b[pl.ds(goff + j * EMB + gg * 16, 16)] = \
                        plsc.load_gather(chk_b, [rows, col])

            @pl.when(is_tail)
            def _(idx=idx, j=j):
                row = idx - _TAIL_LO
                for gg in range(4):
                    stg_a[pl.ds(goff + j * EMB + gg * 16, 16)] = \
                        tail_a[row, pl.ds(gg * 16, 16)]
                    stg_b[pl.ds(goff + j * EMB + gg * 16, 16)] = \
                        tail_b[row, pl.ds(gg * 16, 16)]

            dst = pl.multiple_of(pos * EMB, EMB)
            pltpu.async_copy(stg_a.at[pl.ds(goff + j * EMB, EMB)],
                             out_a_hbm.at[pl.ds(dst, EMB)], sem_oa)
            pltpu.async_copy(stg_b.at[pl.ds(goff + j * EMB, EMB)],
                             out_b_hbm.at[pl.ds(dst, EMB)], sem_ob)
            cur = lax.select(is_tail, cur, c)
        return cur

    lax.fori_loop(0, _BPW // 16, group_body, jnp.int32(-1))

    pltpu.make_async_copy(out_a_hbm.at[pl.ds(0, _BPW * EMB)], stg_a,
                          sem_oa).wait()
    pltpu.make_async_copy(out_a_hbm.at[pl.ds(0, _BPW * EMB)], stg_b,
                          sem_ob).wait()


_flat_sds = jax.ShapeDtypeStruct((B * EMB,), jnp.float32)

_sc_gather = pl.kernel(
    _sc_gather_body,
    out_type=[_flat_sds, _flat_sds],
    mesh=plsc.VectorSubcoreMesh(core_axis_name="c", subcore_axis_name="s"),
    scratch_types=[
        pltpu.VMEM((_BPW,), jnp.int32),
        pltpu.VMEM((_BPW,), jnp.int32),
        pltpu.VMEM((EMB, _CW), jnp.float32),
        pltpu.VMEM((EMB, _CW), jnp.float32),
        pltpu.VMEM((_NTAIL, EMB), jnp.float32),
        pltpu.VMEM((_NTAIL, EMB), jnp.float32),
        pltpu.VMEM((_BPW * EMB,), jnp.float32),
        pltpu.VMEM((_BPW * EMB,), jnp.float32),
        pltpu.SemaphoreType.DMA,
        pltpu.SemaphoreType.DMA,
        pltpu.SemaphoreType.DMA,
        pltpu.SemaphoreType.DMA,
    ],
    compiler_params=pltpu.CompilerParams(use_tc_tiling_on_sc=True,
                                         needs_layout_passes=False),
)


_CB = 2048
_TB = 2048
_HI = jax.lax.Precision.HIGHEST


def _champ_kernel(ids_ref, gmf_ceT, mlp_ceT, gc_ref, xc_ref):
    ids = ids_ref[...]
    champ = lax.broadcasted_iota(jnp.int32, (_CB, NCHAMPS), 1)
    onehot = jnp.where(champ == ids, 1.0, 0.0)
    dn = (((1,), (1,)), ((), ()))
    gc_ref[...] = lax.dot_general(onehot, gmf_ceT[...], dn,
                                  preferred_element_type=jnp.float32,
                                  precision=_HI)
    xc_ref[...] = lax.dot_general(onehot, mlp_ceT[...], dn,
                                  preferred_element_type=jnp.float32,
                                  precision=_HI)


def _dense_kernel(gmf_p, gmf_c, mlp_p, mlp_c, w1t, b1, w2t, b2, w3t, b3,
                  wp_g, wp_h, bp, out_ref):
    f32 = jnp.float32
    h = jnp.dot(mlp_p[...], w1t[:EMB, :], preferred_element_type=f32,
                precision=_HI)
    h += jnp.dot(mlp_c[...], w1t[EMB:, :], preferred_element_type=f32,
                 precision=_HI)
    h = jnp.maximum(h + b1[...], 0.0)
    h = jnp.maximum(
        jnp.dot(h, w2t[...], preferred_element_type=f32, precision=_HI)
        + b2[...], 0.0)
    h = jnp.maximum(
        jnp.dot(h, w3t[...], preferred_element_type=f32, precision=_HI)
        + b3[...], 0.0)
    g = gmf_p[...] * gmf_c[...]
    logit = jnp.dot(g, wp_g[...], preferred_element_type=f32, precision=_HI)
    logit += jnp.dot(h, wp_h[...], preferred_element_type=f32, precision=_HI)
    out_ref[...] = jax.nn.sigmoid(logit + bp[...])


def _full(shape):
    return pl.BlockSpec(shape, lambda i: tuple(0 for _ in shape))


def kernel(player_ids, champion_ids, gmf_pe, gmf_ce, mlp_pe, mlp_ce,
           W1, b1, W2, b2, W3, b3, Wp, bp):
    ids_p = player_ids.astype(jnp.int32)
    ids_c = champion_ids.astype(jnp.int32)

    gmf_peT = gmf_pe.T
    mlp_peT = mlp_pe.T
    gmf_ceT = gmf_ce.T
    mlp_ceT = mlp_ce.T

    gmf_tail = gmf_pe[_TAIL_LO:, :]
    mlp_tail = mlp_pe[_TAIL_LO:, :]

    sorted_ids, order = lax.sort_key_val(
        ids_p, lax.iota(jnp.int32, B))

    gmf_p_flat, mlp_p_flat = _sc_gather(
        sorted_ids, order, gmf_peT, mlp_peT, gmf_tail, mlp_tail)
    gmf_p = gmf_p_flat.reshape(B, EMB)
    mlp_p = mlp_p_flat.reshape(B, EMB)

    ids_c_col = ids_c.reshape(B, 1)
    row_spec = pl.BlockSpec((_CB, EMB), lambda i: (i, 0))
    gmf_c, mlp_c = pl.pallas_call(
        _champ_kernel,
        grid=(B // _CB,),
        in_specs=[
            pl.BlockSpec((_CB, 1), lambda i: (i, 0)),
            _full((EMB, NCHAMPS)), _full((EMB, NCHAMPS)),
        ],
        out_specs=[row_spec, row_spec],
        out_shape=[jax.ShapeDtypeStruct((B, EMB), jnp.float32)] * 2,
    )(ids_c_col, gmf_ceT, mlp_ceT)

    row_spec2 = pl.BlockSpec((_TB, EMB), lambda i: (i, 0))
    out = pl.pallas_call(
        _dense_kernel,
        grid=(B // _TB,),
        in_specs=[
            row_spec2, row_spec2, row_spec2, row_spec2,
            _full((128, 128)), _full((1, 128)),
            _full((128, 64)), _full((1, 64)),
            _full((64, 32)), _full((1, 32)),
            _full((64, 1)), _full((32, 1)), _full((1, 1)),
        ],
        out_specs=pl.BlockSpec((_TB, 1), lambda i: (i, 0)),
        out_shape=jax.ShapeDtypeStruct((B, 1), jnp.float32),
    )(gmf_p, gmf_c, mlp_p, mlp_c,
      W1.T, b1.reshape(1, 128), W2.T, b2.reshape(1, 64),
      W3.T, b3.reshape(1, 32),
      Wp[:, :EMB].T, Wp[:, EMB:].T, bp.reshape(1, 1))
    return out


# speedup vs baseline: 1.3227x; 1.0640x over previous
import jax
import jax.numpy as jnp
from jax import lax
from jax.experimental import pallas as pl
from jax.experimental.pallas import tpu as pltpu
from jax.experimental.pallas import tpu_sc as plsc

B = 16384
EMB = 64
NPLAYERS = 1000000
NCHAMPS = 1000

_SC_INFO = plsc.get_sparse_core_info()
_NC = _SC_INFO.num_cores
_NS = _SC_INFO.num_subcores
_NW = _NC * _NS
_BPW = B // _NW

_CW = 256
_TAIL_LO = (NPLAYERS // _CW) * _CW
_NTAIL = NPLAYERS - _TAIL_LO
_BIG = 0x3FFFFFFF


def _sc_gather_body(sorted_hbm, order_hbm, gmf_peT, mlp_peT,
                    gmf_tail_hbm, mlp_tail_hbm,
                    out_a_hbm, out_b_hbm,
                    idx_v, ord_v,
                    chk_a, chk_b, tail_a, tail_b, stg_a, stg_b,
                    sem_a, sem_b, sem_oa, sem_ob):
    wid = lax.axis_index("s") * _NC + lax.axis_index("c")
    base = wid * _BPW
    pltpu.sync_copy(sorted_hbm.at[pl.ds(base, _BPW)], idx_v)
    pltpu.sync_copy(order_hbm.at[pl.ds(base, _BPW)], ord_v)
    pltpu.sync_copy(gmf_tail_hbm, tail_a)
    pltpu.sync_copy(mlp_tail_hbm, tail_b)

    lanes = lax.iota(jnp.int32, 16)
    chunk0 = gmf_peT.at[:, pl.ds(0, _CW)]

    def group_body(g, cur):
        ids16 = idx_v[pl.ds(g * 16, 16)]
        ord16 = ord_v[pl.ds(g * 16, 16)]
        goff = lax.rem(g, _BPW // 32) * 16 * EMB
        for j in range(16):
            idx = ids16[j]
            pos = ord16[j]
            is_tail = idx >= _TAIL_LO
            c = lax.shift_right_logical(idx, 8)
            need = jnp.logical_and(jnp.logical_not(is_tail), c != cur)

            @pl.when(need)
            def _(c=c):
                cc = pl.multiple_of(c * _CW, _CW)
                pltpu.async_copy(gmf_peT.at[:, pl.ds(cc, _CW)], chk_a, sem_a)
                pltpu.async_copy(mlp_peT.at[:, pl.ds(cc, _CW)], chk_b, sem_b)
                pltpu.make_async_copy(chunk0, chk_a, sem_a).wait()
                pltpu.make_async_copy(chunk0, chk_b, sem_b).wait()

            @pl.when(jnp.logical_not(is_tail))
            def _(idx=idx, c=c, j=j):
                col = jnp.full((16,), idx - c * _CW, dtype=jnp.int32)
                for gg in range(4):
                    rows = lanes + (gg * 16)
                    stg_a[pl.ds(goff + j * EMB + gg * 16, 16)] = \
                        plsc.load_gather(chk_a, [rows, col])
                    stg_b[pl.ds(goff + j * EMB + gg * 16, 16)] = \
                        plsc.load_gather(chk_b, [rows, col])

            @pl.when(is_tail)
            def _(idx=idx, j=j):
                row = idx - _TAIL_LO
                for gg in range(4):
                    stg_a[pl.ds(goff + j * EMB + gg * 16, 16)] = \
                        tail_a[row, pl.ds(gg * 16, 16)]
                    stg_b[pl.ds(goff + j * EMB + gg * 16, 16)] = \
                        tail_b[row, pl.ds(gg * 16, 16)]

            dst = pl.multiple_of(pos * EMB, EMB)
            pltpu.async_copy(stg_a.at[pl.ds(goff + j * EMB, EMB)],
                             out_a_hbm.at[pl.ds(dst, EMB)], sem_oa)
            pltpu.async_copy(stg_b.at[pl.ds(goff + j * EMB, EMB)],
                             out_b_hbm.at[pl.ds(dst, EMB)], sem_ob)
            cur = lax.select(is_tail, cur, c)
        return cur

    cur = lax.fori_loop(0, _BPW // 32, group_body, jnp.int32(-1))
    half = out_a_hbm.at[pl.ds(0, _BPW * EMB // 2)]
    pltpu.make_async_copy(half, stg_a, sem_oa).wait()
    pltpu.make_async_copy(half, stg_b, sem_ob).wait()
    lax.fori_loop(_BPW // 32, _BPW // 16, group_body, cur)
    pltpu.make_async_copy(half, stg_a, sem_oa).wait()
    pltpu.make_async_copy(half, stg_b, sem_ob).wait()


_flat_sds = jax.ShapeDtypeStruct((B * EMB,), jnp.float32)

_sc_gather = pl.kernel(
    _sc_gather_body,
    out_type=[_flat_sds, _flat_sds],
    mesh=plsc.VectorSubcoreMesh(core_axis_name="c", subcore_axis_name="s"),
    scratch_types=[
        pltpu.VMEM((_BPW,), jnp.int32),
        pltpu.VMEM((_BPW,), jnp.int32),
        pltpu.VMEM((EMB, _CW), jnp.float32),
        pltpu.VMEM((EMB, _CW), jnp.float32),
        pltpu.VMEM((_NTAIL, EMB), jnp.float32),
        pltpu.VMEM((_NTAIL, EMB), jnp.float32),
        pltpu.VMEM((_BPW * EMB // 2,), jnp.float32),
        pltpu.VMEM((_BPW * EMB // 2,), jnp.float32),
        pltpu.SemaphoreType.DMA,
        pltpu.SemaphoreType.DMA,
        pltpu.SemaphoreType.DMA,
        pltpu.SemaphoreType.DMA,
    ],
    compiler_params=pltpu.CompilerParams(use_tc_tiling_on_sc=True,
                                         needs_layout_passes=False),
)


_CB = 2048
_TB = 2048
_HI = jax.lax.Precision.HIGHEST


def _champ_kernel(ids_ref, gmf_ceT, mlp_ceT, gc_ref, xc_ref):
    ids = ids_ref[...]
    champ = lax.broadcasted_iota(jnp.int32, (_CB, NCHAMPS), 1)
    onehot = jnp.where(champ == ids, 1.0, 0.0)
    dn = (((1,), (1,)), ((), ()))
    gc_ref[...] = lax.dot_general(onehot, gmf_ceT[...], dn,
                                  preferred_element_type=jnp.float32,
                                  precision=_HI)
    xc_ref[...] = lax.dot_general(onehot, mlp_ceT[...], dn,
                                  preferred_element_type=jnp.float32,
                                  precision=_HI)


def _dense_kernel(gmf_p, gmf_c, mlp_p, mlp_c, w1t, b1, w2t, b2, w3t, b3,
                  wp_g, wp_h, bp, out_ref):
    f32 = jnp.float32
    h = jnp.dot(mlp_p[...], w1t[:EMB, :], preferred_element_type=f32,
                precision=_HI)
    h += jnp.dot(mlp_c[...], w1t[EMB:, :], preferred_element_type=f32,
                 precision=_HI)
    h = jnp.maximum(h + b1[...], 0.0)
    h = jnp.maximum(
        jnp.dot(h, w2t[...], preferred_element_type=f32, precision=_HI)
        + b2[...], 0.0)
    h = jnp.maximum(
        jnp.dot(h, w3t[...], preferred_element_type=f32, precision=_HI)
        + b3[...], 0.0)
    g = gmf_p[...] * gmf_c[...]
    logit = jnp.dot(g, wp_g[...], preferred_element_type=f32, precision=_HI)
    logit += jnp.dot(h, wp_h[...], preferred_element_type=f32, precision=_HI)
    out_ref[...] = jax.nn.sigmoid(logit + bp[...])


def _full(shape):
    return pl.BlockSpec(shape, lambda i: tuple(0 for _ in shape))


def kernel(player_ids, champion_ids, gmf_pe, gmf_ce, mlp_pe, mlp_ce,
           W1, b1, W2, b2, W3, b3, Wp, bp):
    ids_p = player_ids.astype(jnp.int32)
    ids_c = champion_ids.astype(jnp.int32)

    gmf_peT = gmf_pe.T
    mlp_peT = mlp_pe.T
    gmf_ceT = gmf_ce.T
    mlp_ceT = mlp_ce.T

    gmf_tail = gmf_pe[_TAIL_LO:, :]
    mlp_tail = mlp_pe[_TAIL_LO:, :]

    sorted_ids, order = lax.sort_key_val(
        ids_p, lax.iota(jnp.int32, B))

    gmf_p_flat, mlp_p_flat = _sc_gather(
        sorted_ids, order, gmf_peT, mlp_peT, gmf_tail, mlp_tail)
    gmf_p = gmf_p_flat.reshape(B, EMB)
    mlp_p = mlp_p_flat.reshape(B, EMB)

    ids_c_col = ids_c.reshape(B, 1)
    row_spec = pl.BlockSpec((_CB, EMB), lambda i: (i, 0))
    gmf_c, mlp_c = pl.pallas_call(
        _champ_kernel,
        grid=(B // _CB,),
        in_specs=[
            pl.BlockSpec((_CB, 1), lambda i: (i, 0)),
            _full((EMB, NCHAMPS)), _full((EMB, NCHAMPS)),
        ],
        out_specs=[row_spec, row_spec],
        out_shape=[jax.ShapeDtypeStruct((B, EMB), jnp.float32)] * 2,
    )(ids_c_col, gmf_ceT, mlp_ceT)

    row_spec2 = pl.BlockSpec((_TB, EMB), lambda i: (i, 0))
    out = pl.pallas_call(
        _dense_kernel,
        grid=(B // _TB,),
        in_specs=[
            row_spec2, row_spec2, row_spec2, row_spec2,
            _full((128, 128)), _full((1, 128)),
            _full((128, 64)), _full((1, 64)),
            _full((64, 32)), _full((1, 32)),
            _full((64, 1)), _full((32, 1)), _full((1, 1)),
        ],
        out_specs=pl.BlockSpec((_TB, 1), lambda i: (i, 0)),
        out_shape=jax.ShapeDtypeStruct((B, 1), jnp.float32),
    )(gmf_p, gmf_c, mlp_p, mlp_c,
      W1.T, b1.reshape(1, 128), W2.T, b2.reshape(1, 64),
      W3.T, b3.reshape(1, 32),
      Wp[:, :EMB].T, Wp[:, EMB:].T, bp.reshape(1, 1))
    return out
